# speedup vs baseline: 11.6533x; 11.6533x over previous
import functools
import jax
import jax.numpy as jnp
from jax.experimental import pallas as pl
from jax.experimental.pallas import tpu as pltpu

D_EMBED = 2048
N_EXP = 16
CAP = 512
N_BATCH = 2
S_SEQ = 4096

ROW_TILE = 512


def _router_body(x_ref, wt_ref, logits_ref, probs_ref):
    l = jnp.dot(x_ref[...], wt_ref[...], preferred_element_type=jnp.float32)
    m = jnp.max(l, axis=-1, keepdims=True)
    e = jnp.exp(l - m)
    p = e / jnp.sum(e, axis=-1, keepdims=True)
    logits_ref[...] = l
    probs_ref[...] = p


def _mask_body(probs_ref, mask_ref):
    p = probs_ref[...]
    bits = jax.lax.bitcast_convert_type(p, jnp.int32)

    lo0 = jnp.zeros((1, 1, N_EXP), jnp.int32)
    hi0 = jnp.full((1, 1, N_EXP), 0x3F800001, jnp.int32)

    def step(_, lohi):
        lo, hi = lohi
        mid = lo + (hi - lo) // 2
        cnt = jnp.sum((bits >= mid).astype(jnp.int32), axis=1, keepdims=True)
        ge = cnt >= CAP
        return (jnp.where(ge, mid, lo), jnp.where(ge, hi, mid))

    lo, hi = jax.lax.fori_loop(0, 31, step, (lo0, hi0))
    t = lo

    gt = bits > t
    n_gt = jnp.sum(gt.astype(jnp.int32), axis=1, keepdims=True)
    rem = CAP - n_gt

    eq = (bits == t).astype(jnp.int32)
    pref = eq
    sh = 1
    while sh < S_SEQ:
        shifted = jnp.pad(pref, ((0, 0), (sh, 0), (0, 0)))[:, :S_SEQ, :]
        pref = pref + shifted
        sh *= 2
    take_eq = (eq > 0) & (pref <= rem)

    sel = gt | take_eq
    any_sel = jnp.any(sel, axis=-1, keepdims=True)
    col = jax.lax.broadcasted_iota(jnp.int32, (1, 1, N_EXP), 2)
    mask_ref[...] = jnp.where((col == 0) & any_sel, 1.0, 0.0).astype(jnp.float32)


@jax.jit
def kernel(x, W):
    xr = x.reshape(N_BATCH * S_SEQ, D_EMBED)
    wt = W.T

    n_tiles = (N_BATCH * S_SEQ) // ROW_TILE
    logits_r, probs_r = pl.pallas_call(
        _router_body,
        grid=(n_tiles,),
        in_specs=[
            pl.BlockSpec((ROW_TILE, D_EMBED), lambda i: (i, 0)),
            pl.BlockSpec((D_EMBED, N_EXP), lambda i: (0, 0)),
        ],
        out_specs=[
            pl.BlockSpec((ROW_TILE, N_EXP), lambda i: (i, 0)),
            pl.BlockSpec((ROW_TILE, N_EXP), lambda i: (i, 0)),
        ],
        out_shape=[
            jax.ShapeDtypeStruct((N_BATCH * S_SEQ, N_EXP), jnp.float32),
            jax.ShapeDtypeStruct((N_BATCH * S_SEQ, N_EXP), jnp.float32),
        ],
    )(xr, wt)

    logits = logits_r.reshape(N_BATCH, S_SEQ, N_EXP)
    probs = probs_r.reshape(N_BATCH, S_SEQ, N_EXP)

    mask = pl.pallas_call(
        _mask_body,
        grid=(N_BATCH,),
        in_specs=[pl.BlockSpec((1, S_SEQ, N_EXP), lambda b: (b, 0, 0))],
        out_specs=pl.BlockSpec((1, S_SEQ, N_EXP), lambda b: (b, 0, 0)),
        out_shape=jax.ShapeDtypeStruct((N_BATCH, S_SEQ, N_EXP), jnp.float32),
    )(probs)

    return (mask, probs, logits)
